# baseline (speedup 1.0000x reference)
import functools

import jax
import jax.numpy as jnp
from jax import lax
from jax.experimental import pallas as pl
from jax.experimental.pallas import tpu as pltpu
from jax.experimental.pallas import tpu_sc as plsc

VOCAB = 256
HIDDEN = 32
LLM_DIM = 768
LN_EPS = 1e-5


def _table_body(e_ref, w_ref, b_ref, g_ref, be_ref, q_ref):
    y = lax.dot_general(
        e_ref[...], w_ref[...], (((1,), (1,)), ((), ())),
        preferred_element_type=jnp.float32,
        precision=lax.Precision.HIGHEST,
    )
    y = y + b_ref[...]
    mean = jnp.mean(y, axis=1, keepdims=True)
    ctr = y - mean
    var = jnp.mean(ctr * ctr, axis=1, keepdims=True)
    q_ref[...] = ctr * lax.rsqrt(var + LN_EPS) * g_ref[...] + be_ref[...]


def _build_table(emb_table, W, b, gamma, beta):
    return pl.pallas_call(
        _table_body,
        out_shape=jax.ShapeDtypeStruct((VOCAB, LLM_DIM), jnp.float32),
    )(emb_table, W, b.reshape(1, LLM_DIM), gamma.reshape(1, LLM_DIM),
      beta.reshape(1, LLM_DIM))


_NC, _NS = 2, 16
_NW = _NC * _NS
_NG = _NW // 2
_HALF = LLM_DIM // 2
_CHUNK = 16
_NSLOT = 4
_IDXBUF = 3200
_LANES = 16


def _make_gather(n_tok):
    per_g = n_tok // _NG
    n_chunks = per_g // _CHUNK
    n_blocks = per_g // _IDXBUF
    ch_per_blk = _IDXBUF // _CHUNK
    mesh = plsc.VectorSubcoreMesh(core_axis_name="c", subcore_axis_name="s")

    @functools.partial(
        pl.kernel,
        out_type=jax.ShapeDtypeStruct((n_tok, LLM_DIM), jnp.float32),
        mesh=mesh,
        scratch_types=[
            pltpu.VMEM((VOCAB, _HALF), jnp.float32),
            pltpu.VMEM((_NSLOT, _CHUNK, _HALF), jnp.float32),
            pltpu.VMEM((2, _IDXBUF), jnp.int32),
            pltpu.SemaphoreType.DMA,
            pltpu.SemaphoreType.DMA,
        ],
    )
    def gather(q_hbm, idx_hbm, out_hbm, q_v, stage_v, idx_v, ssem, isem):
        wid = lax.axis_index("s") * _NC + lax.axis_index("c")
        g = wid // 2
        h = wid % 2
        col0 = h * _HALF
        tok0 = g * per_g

        def idxcp(blk, bslot):
            return pltpu.make_async_copy(
                idx_hbm.at[pl.ds(tok0 + blk * _IDXBUF, _IDXBUF)],
                idx_v.at[bslot], isem)

        idxcp(0, 0).start()
        pltpu.sync_copy(q_hbm.at[:, pl.ds(col0, _HALF)], q_v)

        def store(c, slot):
            return pltpu.make_async_copy(
                stage_v.at[slot],
                out_hbm.at[pl.ds(tok0 + c * _CHUNK, _CHUNK),
                           pl.ds(col0, _HALF)], ssem)

        def blk_body(blk, carry):
            bslot = lax.rem(blk, 2)
            idxcp(blk, bslot).wait()
            @pl.when(blk + 1 < n_blocks)
            def _prefetch_idx():
                idxcp(blk + 1, 1 - bslot).start()

            def ch_body(j, carry2):
                c = blk * ch_per_blk + j
                slot = lax.rem(j, _NSLOT)
                @pl.when(c >= _NSLOT)
                def _free_slot():
                    store(c - _NSLOT, slot).wait()
                nk = _HALF // _LANES
                ivecs = [idx_v[bslot, pl.ds(j * _CHUNK + tv * _LANES, _LANES)]
                         for tv in range(_CHUNK // _LANES)]

                def load_row(tok):
                    i = ivecs[tok // _LANES][tok % _LANES]
                    return [q_v[i, pl.ds(k * _LANES, _LANES)]
                            for k in range(nk)]

                row = load_row(0)
                for tok in range(_CHUNK):
                    if tok + 1 < _CHUNK:
                        i = ivecs[(tok + 1) // _LANES][(tok + 1) % _LANES]
                        nxt = []
                        for k in range(nk):
                            nxt.append(q_v[i, pl.ds(k * _LANES, _LANES)])
                            stage_v[slot, tok,
                                    pl.ds(k * _LANES, _LANES)] = row[k]
                    else:
                        nxt = None
                        for k in range(nk):
                            stage_v[slot, tok,
                                    pl.ds(k * _LANES, _LANES)] = row[k]
                    row = nxt
                store(c, slot).start()
                return carry2

            lax.fori_loop(0, ch_per_blk, ch_body, 0, unroll=False)
            return carry

        lax.fori_loop(0, n_blocks, blk_body, 0, unroll=False)

        for c in range(n_chunks - _NSLOT, n_chunks):
            store(c, c % _NSLOT).wait()

    return gather


def kernel(payload_head, emb_table, W, b, gamma, beta):
    B, L = payload_head.shape
    n_tok = B * L
    q = _build_table(emb_table, W, b, gamma, beta)
    idx = payload_head.reshape(n_tok).astype(jnp.int32)
    out = _make_gather(n_tok)(q, idx)
    return out.reshape(B, L, LLM_DIM)


# speedup vs baseline: 1.0414x; 1.0414x over previous
import functools

import jax
import jax.numpy as jnp
from jax import lax
from jax.experimental import pallas as pl
from jax.experimental.pallas import tpu as pltpu
from jax.experimental.pallas import tpu_sc as plsc

VOCAB = 256
HIDDEN = 32
LLM_DIM = 768
LN_EPS = 1e-5


def _table_body(e_ref, w_ref, b_ref, g_ref, be_ref, q_ref):
    y = lax.dot_general(
        e_ref[...], w_ref[...], (((1,), (1,)), ((), ())),
        preferred_element_type=jnp.float32,
        precision=lax.Precision.HIGHEST,
    )
    y = y + b_ref[...]
    mean = jnp.mean(y, axis=1, keepdims=True)
    ctr = y - mean
    var = jnp.mean(ctr * ctr, axis=1, keepdims=True)
    q_ref[...] = ctr * lax.rsqrt(var + LN_EPS) * g_ref[...] + be_ref[...]


def _build_table(emb_table, W, b, gamma, beta):
    return pl.pallas_call(
        _table_body,
        out_shape=jax.ShapeDtypeStruct((VOCAB, LLM_DIM), jnp.float32),
    )(emb_table, W, b.reshape(1, LLM_DIM), gamma.reshape(1, LLM_DIM),
      beta.reshape(1, LLM_DIM))


_NC, _NS = 2, 16
_NW = _NC * _NS
_NG = _NW // 2
_HALF = LLM_DIM // 2
_CHUNK = 32
_IDXBUF = 3200
_LANES = 16


def _make_gather(n_tok):
    per_g = n_tok // _NG
    n_chunks = per_g // _CHUNK
    n_blocks = per_g // _IDXBUF
    ch_per_blk = _IDXBUF // _CHUNK
    mesh = plsc.VectorSubcoreMesh(core_axis_name="c", subcore_axis_name="s")

    @functools.partial(
        pl.kernel,
        out_type=jax.ShapeDtypeStruct((n_tok, LLM_DIM), jnp.float32),
        mesh=mesh,
        scratch_types=[
            pltpu.VMEM((VOCAB, _HALF), jnp.float32),
            pltpu.VMEM((2, _CHUNK, _HALF), jnp.float32),
            pltpu.VMEM((2, _IDXBUF), jnp.int32),
            pltpu.SemaphoreType.DMA,
            pltpu.SemaphoreType.DMA,
        ],
    )
    def gather(q_hbm, idx_hbm, out_hbm, q_v, stage_v, idx_v, ssem, isem):
        wid = lax.axis_index("s") * _NC + lax.axis_index("c")
        g = wid // 2
        h = wid % 2
        col0 = h * _HALF
        tok0 = g * per_g

        def idxcp(blk, bslot):
            return pltpu.make_async_copy(
                idx_hbm.at[pl.ds(tok0 + blk * _IDXBUF, _IDXBUF)],
                idx_v.at[bslot], isem)

        idxcp(0, 0).start()
        pltpu.sync_copy(q_hbm.at[:, pl.ds(col0, _HALF)], q_v)

        def store(c, slot):
            return pltpu.make_async_copy(
                stage_v.at[slot],
                out_hbm.at[pl.ds(tok0 + c * _CHUNK, _CHUNK),
                           pl.ds(col0, _HALF)], ssem)

        def blk_body(blk, carry):
            bslot = lax.rem(blk, 2)
            idxcp(blk, bslot).wait()
            @pl.when(blk + 1 < n_blocks)
            def _prefetch_idx():
                idxcp(blk + 1, 1 - bslot).start()

            def ch_body(j, carry2):
                c = blk * ch_per_blk + j
                slot = lax.rem(j, 2)
                @pl.when(c >= 2)
                def _free_slot():
                    store(c - 2, slot).wait()
                nk = _HALF // _LANES
                ivecs = [idx_v[bslot, pl.ds(j * _CHUNK + tv * _LANES, _LANES)]
                         for tv in range(_CHUNK // _LANES)]

                def load_row(tok):
                    i = ivecs[tok // _LANES][tok % _LANES]
                    return [q_v[i, pl.ds(k * _LANES, _LANES)]
                            for k in range(nk)]

                row = load_row(0)
                for tok in range(_CHUNK):
                    if tok + 1 < _CHUNK:
                        i = ivecs[(tok + 1) // _LANES][(tok + 1) % _LANES]
                        nxt = []
                        for k in range(nk):
                            nxt.append(q_v[i, pl.ds(k * _LANES, _LANES)])
                            stage_v[slot, tok,
                                    pl.ds(k * _LANES, _LANES)] = row[k]
                    else:
                        nxt = None
                        for k in range(nk):
                            stage_v[slot, tok,
                                    pl.ds(k * _LANES, _LANES)] = row[k]
                    row = nxt
                store(c, slot).start()
                return carry2

            lax.fori_loop(0, ch_per_blk, ch_body, 0, unroll=False)
            return carry

        lax.fori_loop(0, n_blocks, blk_body, 0, unroll=False)

        for c in (n_chunks - 2, n_chunks - 1):
            store(c, c % 2).wait()

    return gather


def kernel(payload_head, emb_table, W, b, gamma, beta):
    B, L = payload_head.shape
    n_tok = B * L
    q = _build_table(emb_table, W, b, gamma, beta)
    idx = payload_head.reshape(n_tok).astype(jnp.int32)
    out = _make_gather(n_tok)(q, idx)
    return out.reshape(B, L, LLM_DIM)
